# baseline (speedup 1.0000x reference)
import functools

import jax
import jax.numpy as jnp
from jax import lax
from jax.experimental import pallas as pl
from jax.experimental.pallas import tpu as pltpu
from jax.experimental.pallas import tpu_sc as plsc

ROWS, COLS = 16384, 200
VOCAB, D = 25, 256
DP = D + 1
REP = 4
LANES = 16
CU = 8
B = ROWS * COLS
NC, NS = 2, 16
NW = NC * NS
B_PER_W = B // NW
CHUNK = 80
NCHUNK = B_PER_W // CHUNK
NBUF = 2
OUTER = NCHUNK // NBUF


@functools.partial(
    pl.kernel,
    out_type=jax.ShapeDtypeStruct((B, D), jnp.float32),
    mesh=plsc.VectorSubcoreMesh(
        core_axis_name="c", subcore_axis_name="s", num_cores=NC, num_subcores=NS
    ),
    scratch_types=[
        pltpu.VMEM((NBUF, CHUNK), jnp.int32),
        pltpu.VMEM((NBUF, CHUNK, DP), jnp.float32),
        pltpu.VMEM((VOCAB * REP * DP,), jnp.float32),
    ]
    + [pltpu.SemaphoreType.DMA] * (2 * NBUF),
    compiler_params=pltpu.CompilerParams(needs_layout_passes=False),
)
def _embed_expand(idx_hbm, rep_hbm, out_hbm, idx_v, rows_v, table_f, *sems):
    sem_i = sems[0:NBUF]
    sem_s = sems[NBUF : 2 * NBUF]
    wid = lax.axis_index("s") * NC + lax.axis_index("c")
    base = wid * B_PER_W

    def idx_src(chunk):
        return idx_hbm.at[pl.ds(base + chunk * CHUNK, CHUNK)]

    def out_dst(chunk):
        return out_hbm.at[pl.ds(base + chunk * CHUNK, CHUNK), :]

    def rows_src(b):
        return rows_v.at[b, :, pl.ds(0, D)]

    pltpu.sync_copy(rep_hbm, table_f)

    for b in range(NBUF):
        pltpu.async_copy(idx_src(b), idx_v.at[b], sem_i[b])

    lane_iota = lax.iota(jnp.int32, LANES)
    lane_rep = lax.rem(lane_iota, jnp.full((LANES,), REP, jnp.int32)) * DP

    def outer(t, carry):
        for b in range(NBUF):
            i = t * NBUF + b

            @pl.when(t > 0)
            def _wait_prev_scatter():
                pltpu.make_async_copy(rows_src(b), out_dst(0), sem_s[b]).wait()

            pltpu.make_async_copy(idx_src(0), idx_v.at[b], sem_i[b]).wait()

            def group(g, c2):
                r0 = g * LANES
                src_base = idx_v[b, pl.ds(r0, LANES)] * (REP * DP) + lane_rep
                row_ids = lane_iota + r0
                dst = rows_v.at[b]

                def cols(ci, c3):
                    c0 = ci * CU
                    src_c0 = src_base + c0
                    col_c0 = jnp.zeros((LANES,), jnp.int32) + c0
                    for cc in range(CU):
                        col = plsc.load_gather(table_f, [src_c0 + cc])
                        plsc.store_scatter(dst, [row_ids, col_c0 + cc], col)
                    return c3

                lax.fori_loop(0, D // CU, cols, 0)
                return c2

            lax.fori_loop(0, CHUNK // LANES, group, 0)

            pltpu.async_copy(rows_src(b), out_dst(i), sem_s[b])
            nxt = jnp.minimum(i + NBUF, NCHUNK - 1)
            pltpu.async_copy(idx_src(nxt), idx_v.at[b], sem_i[b])
        return carry

    lax.fori_loop(0, OUTER, outer, 0)

    for b in range(NBUF):
        pltpu.make_async_copy(idx_src(0), idx_v.at[b], sem_i[b]).wait()
        pltpu.make_async_copy(rows_src(b), out_dst(0), sem_s[b]).wait()


def kernel(x, table):
    idx = x.reshape(B)
    padded = jnp.pad(table, ((0, 0), (0, DP - D)))
    rep = jnp.repeat(padded, REP, axis=0).reshape(VOCAB * REP * DP)
    out = _embed_expand(idx, rep)
    return out.reshape(ROWS, COLS, D)


# speedup vs baseline: 2.9401x; 2.9401x over previous
import functools

import jax
import jax.numpy as jnp
from jax import lax
from jax.experimental import pallas as pl
from jax.experimental.pallas import tpu as pltpu
from jax.experimental.pallas import tpu_sc as plsc

ROWS, COLS = 16384, 200
VOCAB, D = 25, 256
LANES = 16
RU = 4
B = ROWS * COLS
NC, NS = 2, 16
NW = NC * NS
B_PER_W = B // NW
CHUNK = 128
NCHUNK = B_PER_W // CHUNK
NBUF = 2
OUTER = NCHUNK // NBUF


@functools.partial(
    pl.kernel,
    out_type=jax.ShapeDtypeStruct((B, D), jnp.float32),
    mesh=plsc.VectorSubcoreMesh(
        core_axis_name="c", subcore_axis_name="s", num_cores=NC, num_subcores=NS
    ),
    scratch_types=[
        pltpu.VMEM((NBUF, CHUNK), jnp.int32),
        pltpu.VMEM((NBUF, CHUNK, D), jnp.float32),
        pltpu.VMEM((VOCAB * D,), jnp.float32),
        pltpu.SMEM((CHUNK,), jnp.int32),
    ]
    + [pltpu.SemaphoreType.DMA] * (2 * NBUF),
)
def _embed_expand(idx_hbm, table_hbm, out_hbm, idx_v, rows_v, table_f, soff,
                  *sems):
    sem_i = sems[0:NBUF]
    sem_s = sems[NBUF : 2 * NBUF]
    wid = lax.axis_index("s") * NC + lax.axis_index("c")
    base = wid * B_PER_W

    def idx_src(chunk):
        return idx_hbm.at[pl.ds(base + chunk * CHUNK, CHUNK)]

    def out_dst(chunk):
        return out_hbm.at[pl.ds(base + chunk * CHUNK, CHUNK), :]

    pltpu.sync_copy(table_hbm, table_f)

    for b in range(NBUF):
        pltpu.async_copy(idx_src(b), idx_v.at[b], sem_i[b])

    def outer(t, carry):
        for b in range(NBUF):
            i = t * NBUF + b

            @pl.when(t > 0)
            def _wait_prev_scatter():
                pltpu.make_async_copy(rows_v.at[b], out_dst(0), sem_s[b]).wait()

            pltpu.make_async_copy(idx_src(0), idx_v.at[b], sem_i[b]).wait()

            def stage(g, c2):
                ivec = idx_v[b, pl.ds(g * LANES, LANES)] * D
                for l in range(LANES):
                    soff[g * LANES + l] = ivec[l]
                return c2

            lax.fori_loop(0, CHUNK // LANES, stage, 0)

            def rows(q, c2):
                for u in range(RU):
                    r = q * RU + u
                    s = soff[r]
                    for c in range(D // LANES):
                        rows_v[b, r, pl.ds(c * LANES, LANES)] = table_f[
                            pl.ds(s + c * LANES, LANES)
                        ]
                return c2

            lax.fori_loop(0, CHUNK // RU, rows, 0)

            pltpu.async_copy(rows_v.at[b], out_dst(i), sem_s[b])
            nxt = jnp.minimum(i + NBUF, NCHUNK - 1)
            pltpu.async_copy(idx_src(nxt), idx_v.at[b], sem_i[b])
        return carry

    lax.fori_loop(0, OUTER, outer, 0)

    for b in range(NBUF):
        pltpu.make_async_copy(idx_src(0), idx_v.at[b], sem_i[b]).wait()
        pltpu.make_async_copy(rows_v.at[b], out_dst(0), sem_s[b]).wait()


def kernel(x, table):
    idx = x.reshape(B)
    out = _embed_expand(idx, table.reshape(VOCAB * D))
    return out.reshape(ROWS, COLS, D)


# speedup vs baseline: 15.0610x; 5.1225x over previous
import functools

import jax
import jax.numpy as jnp
from jax import lax
from jax.experimental import pallas as pl
from jax.experimental.pallas import tpu as pltpu
from jax.experimental.pallas import tpu_sc as plsc

ROWS, COLS = 16384, 200
VOCAB, D = 25, 256
LANES = 16
RU = 4
B = ROWS * COLS
NC, NS = 2, 16
NW = NC * NS
B_PER_W = B // NW
CHUNK = 128
NCHUNK = B_PER_W // CHUNK
NBUF = 2
OUTER = NCHUNK // NBUF


@functools.partial(
    pl.kernel,
    out_type=jax.ShapeDtypeStruct((B, D), jnp.float32),
    mesh=plsc.VectorSubcoreMesh(
        core_axis_name="c", subcore_axis_name="s", num_cores=NC, num_subcores=NS
    ),
    scratch_types=[
        pltpu.VMEM((NBUF, CHUNK), jnp.int32),
        pltpu.VMEM((NBUF, CHUNK, D), jnp.float32),
        pltpu.VMEM((VOCAB * D,), jnp.float32),
        pltpu.SMEM((CHUNK,), jnp.int32),
    ]
    + [pltpu.SemaphoreType.DMA] * (2 * NBUF),
)
def _embed_expand(idx_hbm, table_hbm, out_hbm, idx_v, rows_v, table_f, soff,
                  *sems):
    sem_i = sems[0:NBUF]
    sem_s = sems[NBUF : 2 * NBUF]
    wid = lax.axis_index("s") * NC + lax.axis_index("c")
    base = wid * B_PER_W

    def idx_src(chunk):
        return idx_hbm.at[pl.ds(base + chunk * CHUNK, CHUNK)]

    def out_dst(chunk):
        return out_hbm.at[pl.ds(base + chunk * CHUNK, CHUNK), :]

    pltpu.sync_copy(table_hbm, table_f)

    for b in range(NBUF):
        pltpu.async_copy(idx_src(b), idx_v.at[b], sem_i[b])

    def outer(t, carry):
        for b in range(NBUF):
            i = t * NBUF + b

            @pl.when(t > 0)
            def _wait_prev_scatter():
                pltpu.make_async_copy(rows_v.at[b], out_dst(0), sem_s[b]).wait()

            pltpu.make_async_copy(idx_src(0), idx_v.at[b], sem_i[b]).wait()

            def stage(g, c2):
                ivec = idx_v[b, pl.ds(g * LANES, LANES)] * D
                for l in range(LANES):
                    soff[g * LANES + l] = ivec[l]
                return c2

            lax.fori_loop(0, CHUNK // LANES, stage, 0)

            @functools.partial(plsc.parallel_loop, 0, CHUNK, unroll=RU)
            def rows(r):
                s = soff[r]
                for c in range(D // LANES):
                    rows_v[b, r, pl.ds(c * LANES, LANES)] = table_f[
                        pl.ds(s + c * LANES, LANES)
                    ]

            pltpu.async_copy(rows_v.at[b], out_dst(i), sem_s[b])
            nxt = jnp.minimum(i + NBUF, NCHUNK - 1)
            pltpu.async_copy(idx_src(nxt), idx_v.at[b], sem_i[b])
        return carry

    lax.fori_loop(0, OUTER, outer, 0)

    for b in range(NBUF):
        pltpu.make_async_copy(idx_src(0), idx_v.at[b], sem_i[b]).wait()
        pltpu.make_async_copy(rows_v.at[b], out_dst(0), sem_s[b]).wait()


def kernel(x, table):
    idx = x.reshape(B)
    out = _embed_expand(idx, table.reshape(VOCAB * D))
    return out.reshape(ROWS, COLS, D)
